# baseline (speedup 1.0000x reference)
import functools

import jax
import jax.numpy as jnp
from jax import lax
from jax.experimental import pallas as pl
from jax.experimental.pallas import tpu as pltpu
from jax.experimental.pallas import tpu_sc as plsc

B = 32
SS = 12
NW = 32
TN = 128
BF16 = jnp.bfloat16


def _round_up(v, m):
    return (v + m - 1) // m * m


def _npad(n_v):
    return _round_up(n_v, TN)


def _bgroup(c, o):
    for bg in (8, 16):
        if (bg * c) % 128 == 0 and (bg * o) % 128 == 0:
            return bg
    return B


def _gather_window(bc):
    return max(8, min(128, 32768 // bc // 8 * 8))


def _sc_gather(table, idx):
    bc = table.shape[1]
    m = idx.shape[0]
    win = _gather_window(bc)
    chunk = m // NW
    T = chunk // win
    mesh = plsc.VectorSubcoreMesh(core_axis_name="c", subcore_axis_name="s")

    @functools.partial(
        pl.kernel,
        out_type=jax.ShapeDtypeStruct((m, bc), jnp.float32),
        mesh=mesh,
        scratch_types=[
            pltpu.VMEM((chunk,), jnp.int32),
            pltpu.VMEM((win, bc), jnp.float32),
            pltpu.VMEM((win, bc), jnp.float32),
            pltpu.SemaphoreType.DMA,
            pltpu.SemaphoreType.DMA,
            pltpu.SemaphoreType.DMA,
            pltpu.SemaphoreType.DMA,
            pltpu.SemaphoreType.DMA,
        ],
    )
    def k(tab, idx_hbm, out, idx_v, r0, r1, sl, sg0, sg1, so0, so1):
        w = lax.axis_index("s") * 2 + lax.axis_index("c")
        base = w * chunk
        ld = pltpu.make_async_copy(idx_hbm.at[pl.ds(base, chunk)], idx_v, sl)
        ld.start()
        ld.wait()
        rbufs = (r0, r1)
        gsems = (sg0, sg1)
        osems = (so0, so1)

        def gat(t, b):
            return pltpu.make_async_copy(
                tab.at[idx_v.at[pl.ds(t * win, win)]], rbufs[b], gsems[b])

        def halfstep(t, b):
            @pl.when(jnp.logical_and(t >= 1, t + 1 < T)
                     | (t == T) | jnp.logical_and(t == T + 1, T >= 2))
            def _():
                pltpu.make_async_copy(
                    rbufs[1 - b], out.at[pl.ds(0, win)], osems[1 - b]).wait()

            @pl.when(t + 1 < T)
            def _():
                gat(t + 1, 1 - b).start()

            @pl.when(t < T)
            def _():
                gat(t, b).wait()
                pltpu.make_async_copy(
                    rbufs[b], out.at[pl.ds(base + t * win, win)], osems[b]
                ).start()

        @pl.when(T > 0)
        def _():
            gat(0, 0).start()

        def body(i, carry):
            halfstep(2 * i, 0)
            halfstep(2 * i + 1, 1)
            return carry

        lax.fori_loop(0, (T + 3) // 2, body, 0)

    return k(table, idx)


def _conv_mm(g, wx, bias_bo, n_v, c, o, elu, bg):
    n_pad = _npad(n_v)
    gsz = B // bg
    grid = (n_pad // TN, gsz)
    g3 = g.reshape(SS, n_pad, B * c)

    def body(a_ref, w_ref, b_ref, o_ref):
        acc = jnp.zeros((TN, bg * o), jnp.float32)
        for s in range(SS):
            acc += lax.dot_general(
                a_ref[s].astype(BF16), w_ref[s], (((1,), (0,)), ((), ())),
                preferred_element_type=jnp.float32)
        r = acc + b_ref[...]
        if elu:
            r = jnp.where(r > 0, r, jnp.exp(r) - 1.0)
        i = pl.program_id(0)
        vert = i * TN + lax.broadcasted_iota(jnp.int32, (TN, bg * o), 0)
        o_ref[...] = jnp.where(vert < n_v - 1, r, 0.0)

    return pl.pallas_call(
        body,
        grid=grid,
        in_specs=[
            pl.BlockSpec((SS, TN, bg * c), lambda i, gi: (0, i, gi)),
            pl.BlockSpec((SS, bg * c, bg * o), lambda i, gi: (0, 0, 0)),
            pl.BlockSpec((1, bg * o), lambda i, gi: (0, gi)),
        ],
        out_specs=pl.BlockSpec((TN, bg * o), lambda i, gi: (i, gi)),
        out_shape=jax.ShapeDtypeStruct((n_v, B * o), jnp.float32),
    )(g3, wx, bias_bo)


def _expand_w(wt, bias, c, o, bg):
    wsr = wt.reshape(o, SS, c).transpose(1, 2, 0)
    eye = jnp.eye(bg, dtype=wt.dtype)
    wx = jnp.einsum('bB,sco->sbcBo', eye, wsr)
    wx = wx.reshape(SS, bg * c, bg * o).astype(BF16)
    bias_bo = jnp.tile(bias, B).reshape(1, B * o)
    return wx, bias_bo


def _conv_onehot(h, oh, wx, bias_bo, n_v, c, o, elu, bg):
    n_pad = oh.shape[1]
    gsz = B // bg
    grid = (n_pad // TN, gsz)

    def body(oh_ref, h_ref, w_ref, b_ref, o_ref):
        h16 = h_ref[...].astype(BF16)
        acc = jnp.zeros((TN, bg * o), jnp.float32)
        for s in range(SS):
            g_s = lax.dot_general(
                oh_ref[s], h16, (((1,), (0,)), ((), ())),
                preferred_element_type=jnp.float32)
            acc += lax.dot_general(
                g_s.astype(BF16), w_ref[s], (((1,), (0,)), ((), ())),
                preferred_element_type=jnp.float32)
        r = acc + b_ref[...]
        if elu:
            r = jnp.where(r > 0, r, jnp.exp(r) - 1.0)
        i = pl.program_id(0)
        vert = i * TN + lax.broadcasted_iota(jnp.int32, (TN, bg * o), 0)
        o_ref[...] = jnp.where(vert < n_v - 1, r, 0.0)

    return pl.pallas_call(
        body,
        grid=grid,
        in_specs=[
            pl.BlockSpec((SS, TN, n_v), lambda i, gi: (0, i, 0)),
            pl.BlockSpec((n_v, bg * c), lambda i, gi: (0, gi)),
            pl.BlockSpec((SS, bg * c, bg * o), lambda i, gi: (0, 0, 0)),
            pl.BlockSpec((1, bg * o), lambda i, gi: (0, gi)),
        ],
        out_specs=pl.BlockSpec((TN, bg * o), lambda i, gi: (i, gi)),
        out_shape=jax.ShapeDtypeStruct((n_v, B * o), jnp.float32),
    )(oh, h, wx, bias_bo)


def _sconv(h, idx, wt, bias, n_v, c, o, elu=True):
    bg = _bgroup(c, o)
    wx, bias_bo = _expand_w(wt, bias, c, o, bg)
    if n_v <= 1280:
        return _conv_onehot(h, idx, wx, bias_bo, n_v, c, o, elu, bg)
    g = _sc_gather(h, idx)
    return _conv_mm(g, wx, bias_bo, n_v, c, o, elu, bg)


def _pool(d, hv, tm=256):
    P, Q = d.shape
    BF = hv.shape[1]
    tm = min(tm, _round_up(P, 8))
    grid = (pl.cdiv(P, tm),)

    def body(d_ref, h_ref, o_ref):
        o_ref[...] = lax.dot_general(
            d_ref[...], h_ref[...], (((1,), (0,)), ((), ())),
            preferred_element_type=jnp.float32)

    return pl.pallas_call(
        body,
        grid=grid,
        in_specs=[
            pl.BlockSpec((tm, Q), lambda i: (i, 0)),
            pl.BlockSpec((Q, BF), lambda i: (0, 0)),
        ],
        out_specs=pl.BlockSpec((tm, BF), lambda i: (i, 0)),
        out_shape=jax.ShapeDtypeStruct((P, BF), jnp.float32),
    )(d, hv)


def _mm(a, w, bias):
    M, K = a.shape
    O = w.shape[0]

    def body(a_ref, w_ref, b_ref, o_ref):
        acc = lax.dot_general(
            a_ref[...], w_ref[...], (((1,), (1,)), ((), ())),
            preferred_element_type=jnp.float32)
        o_ref[...] = acc + b_ref[...]

    return pl.pallas_call(
        body,
        in_specs=[pl.BlockSpec((M, K), lambda: (0, 0)),
                  pl.BlockSpec((O, K), lambda: (0, 0)),
                  pl.BlockSpec((1, O), lambda: (0, 0))],
        out_specs=pl.BlockSpec((M, O), lambda: (0, 0)),
        out_shape=jax.ShapeDtypeStruct((M, O), jnp.float32),
    )(a, w, bias.reshape(1, O))


def _mk_idx(spiral, bc, n_v):
    n_pad = _npad(n_v)
    sp = jnp.pad(spiral, ((0, n_pad - n_v), (0, 0)))
    if n_v <= 1280:
        return jax.nn.one_hot(sp.T, n_v, dtype=BF16)
    flat = sp.T.reshape(-1)
    m_pad = _round_up(flat.size, NW * _gather_window(bc))
    return jnp.pad(flat, (0, m_pad - flat.size))


def kernel(x, spiral0, spiral1, spiral2, spiral3,
           Wc0, bc0, Wc1, bc1, Wc2, bc2, Wc3, bc3, Wc4, bc4,
           We, be, Wdfc, bdfc,
           Wd0, bd0, Wd1, bd1, Wd2, bd2, Wd3, bd3, Wd4, bd4,
           D0, D1, D2, D3, U0, U1, U2, U3):
    n0, n1, n2, n3, n4 = 5024, 1257, 315, 80, 21
    sp = (spiral0, spiral1, spiral2, spiral3)
    nv = (n0, n1, n2, n3)
    idx = {}

    def sconv(h, lvl, wt, bias, c, o, elu=True):
        key = (lvl, c)
        if key not in idx:
            idx[key] = _mk_idx(sp[lvl], B * c, nv[lvl])
        return _sconv(h, idx[key], wt, bias, nv[lvl], c, o, elu)

    xp = jnp.pad(x, ((0, 0), (0, 0), (0, 1)))
    h = xp.transpose(1, 0, 2).reshape(n0, B * 4)
    w0 = jnp.pad(Wc0.reshape(16, SS, 3), ((0, 0), (0, 0), (0, 1))).reshape(16, SS * 4)
    h = sconv(h, 0, w0, bc0, 4, 16)
    h = sconv(h, 0, Wc1, bc1, 16, 32)
    h = _pool(D0, h)
    h = sconv(h, 1, Wc2, bc2, 32, 64)
    h = _pool(D1, h)
    h = sconv(h, 2, Wc3, bc3, 64, 96)
    h = _pool(D2, h)
    h = sconv(h, 3, Wc4, bc4, 96, 128)
    h = _pool(D3, h)

    hz = h.reshape(n4, B, 128).transpose(1, 0, 2).reshape(B, n4 * 128)
    z = _mm(hz, We, be)
    hd = _mm(z, Wdfc, bdfc)
    h = hd.reshape(B, n4, 128).transpose(1, 0, 2).reshape(n4, B * 128)

    h = _pool(U3, h)
    h = sconv(h, 3, Wd0, bd0, 128, 96)
    h = _pool(U2, h)
    h = sconv(h, 2, Wd1, bd1, 96, 64)
    h = _pool(U1, h)
    h = sconv(h, 1, Wd2, bd2, 64, 32)
    h = _pool(U0, h)
    h = sconv(h, 0, Wd3, bd3, 32, 32)
    h = sconv(h, 0, Wd4, bd4, 32, 3, elu=False)

    return h.reshape(n0, B, 3).transpose(1, 0, 2)


# speedup vs baseline: 1.0129x; 1.0129x over previous
import functools

import jax
import jax.numpy as jnp
from jax import lax
from jax.experimental import pallas as pl
from jax.experimental.pallas import tpu as pltpu
from jax.experimental.pallas import tpu_sc as plsc

B = 32
SS = 12
NW = 32
TN = 128
BF16 = jnp.bfloat16


def _round_up(v, m):
    return (v + m - 1) // m * m


def _npad(n_v):
    return _round_up(n_v, TN)


def _bgroup(c, o):
    for bg in (8, 16):
        if (bg * c) % 128 == 0 and (bg * o) % 128 == 0:
            return bg
    return B


def _gather_window(bc):
    return max(8, min(128, 49152 // bc // 8 * 8))


def _sc_gather(table, idx):
    bc = table.shape[1]
    m = idx.shape[0]
    win = _gather_window(bc)
    chunk = m // NW
    T = chunk // win
    mesh = plsc.VectorSubcoreMesh(core_axis_name="c", subcore_axis_name="s")

    @functools.partial(
        pl.kernel,
        out_type=jax.ShapeDtypeStruct((m, bc), jnp.float32),
        mesh=mesh,
        scratch_types=[
            pltpu.VMEM((chunk,), jnp.int32),
            pltpu.VMEM((win, bc), jnp.float32),
            pltpu.VMEM((win, bc), jnp.float32),
            pltpu.SemaphoreType.DMA,
            pltpu.SemaphoreType.DMA,
            pltpu.SemaphoreType.DMA,
            pltpu.SemaphoreType.DMA,
            pltpu.SemaphoreType.DMA,
        ],
    )
    def k(tab, idx_hbm, out, idx_v, r0, r1, sl, sg0, sg1, so0, so1):
        w = lax.axis_index("s") * 2 + lax.axis_index("c")
        base = w * chunk
        ld = pltpu.make_async_copy(idx_hbm.at[pl.ds(base, chunk)], idx_v, sl)
        ld.start()
        ld.wait()
        rbufs = (r0, r1)
        gsems = (sg0, sg1)
        osems = (so0, so1)

        def gat(t, b):
            return pltpu.make_async_copy(
                tab.at[idx_v.at[pl.ds(t * win, win)]], rbufs[b], gsems[b])

        def halfstep(t, b):
            @pl.when(jnp.logical_and(t >= 1, t + 1 < T)
                     | (t == T) | jnp.logical_and(t == T + 1, T >= 2))
            def _():
                pltpu.make_async_copy(
                    rbufs[1 - b], out.at[pl.ds(0, win)], osems[1 - b]).wait()

            @pl.when(t + 1 < T)
            def _():
                gat(t + 1, 1 - b).start()

            @pl.when(t < T)
            def _():
                gat(t, b).wait()
                pltpu.make_async_copy(
                    rbufs[b], out.at[pl.ds(base + t * win, win)], osems[b]
                ).start()

        @pl.when(T > 0)
        def _():
            gat(0, 0).start()

        def body(i, carry):
            halfstep(2 * i, 0)
            halfstep(2 * i + 1, 1)
            return carry

        lax.fori_loop(0, (T + 3) // 2, body, 0)

    return k(table, idx)


def _conv_mm(g, wx, bias_bo, n_v, c, o, elu, bg):
    n_pad = _npad(n_v)
    gsz = B // bg
    grid = (n_pad // TN, gsz)
    g3 = g.reshape(SS, n_pad, B * c)

    def body(a_ref, w_ref, b_ref, o_ref):
        acc = jnp.zeros((TN, bg * o), jnp.float32)
        for s in range(SS):
            acc += lax.dot_general(
                a_ref[s].astype(BF16), w_ref[s], (((1,), (0,)), ((), ())),
                preferred_element_type=jnp.float32)
        r = acc + b_ref[...]
        if elu:
            r = jnp.where(r > 0, r, jnp.exp(r) - 1.0)
        i = pl.program_id(0)
        vert = i * TN + lax.broadcasted_iota(jnp.int32, (TN, bg * o), 0)
        o_ref[...] = jnp.where(vert < n_v - 1, r, 0.0)

    return pl.pallas_call(
        body,
        grid=grid,
        in_specs=[
            pl.BlockSpec((SS, TN, bg * c), lambda i, gi: (0, i, gi)),
            pl.BlockSpec((SS, bg * c, bg * o), lambda i, gi: (0, 0, 0)),
            pl.BlockSpec((1, bg * o), lambda i, gi: (0, gi)),
        ],
        out_specs=pl.BlockSpec((TN, bg * o), lambda i, gi: (i, gi)),
        out_shape=jax.ShapeDtypeStruct((n_v, B * o), jnp.float32),
    )(g3, wx, bias_bo)


def _expand_w(wt, bias, c, o, bg):
    wsr = wt.reshape(o, SS, c).transpose(1, 2, 0)
    eye = jnp.eye(bg, dtype=wt.dtype)
    wx = jnp.einsum('bB,sco->sbcBo', eye, wsr)
    wx = wx.reshape(SS, bg * c, bg * o).astype(BF16)
    bias_bo = jnp.tile(bias, B).reshape(1, B * o)
    return wx, bias_bo


def _conv_onehot(h, oh, wx, bias_bo, n_v, c, o, elu, bg):
    n_pad = oh.shape[1]
    gsz = B // bg
    grid = (n_pad // TN, gsz)

    def body(oh_ref, h_ref, w_ref, b_ref, o_ref):
        h16 = h_ref[...].astype(BF16)
        acc = jnp.zeros((TN, bg * o), jnp.float32)
        for s in range(SS):
            g_s = lax.dot_general(
                oh_ref[s], h16, (((1,), (0,)), ((), ())),
                preferred_element_type=jnp.float32)
            acc += lax.dot_general(
                g_s.astype(BF16), w_ref[s], (((1,), (0,)), ((), ())),
                preferred_element_type=jnp.float32)
        r = acc + b_ref[...]
        if elu:
            r = jnp.where(r > 0, r, jnp.exp(r) - 1.0)
        i = pl.program_id(0)
        vert = i * TN + lax.broadcasted_iota(jnp.int32, (TN, bg * o), 0)
        o_ref[...] = jnp.where(vert < n_v - 1, r, 0.0)

    return pl.pallas_call(
        body,
        grid=grid,
        in_specs=[
            pl.BlockSpec((SS, TN, n_v), lambda i, gi: (0, i, 0)),
            pl.BlockSpec((n_v, bg * c), lambda i, gi: (0, gi)),
            pl.BlockSpec((SS, bg * c, bg * o), lambda i, gi: (0, 0, 0)),
            pl.BlockSpec((1, bg * o), lambda i, gi: (0, gi)),
        ],
        out_specs=pl.BlockSpec((TN, bg * o), lambda i, gi: (i, gi)),
        out_shape=jax.ShapeDtypeStruct((n_v, B * o), jnp.float32),
    )(oh, h, wx, bias_bo)


def _sconv(h, idx, wt, bias, n_v, c, o, elu=True):
    bg = _bgroup(c, o)
    wx, bias_bo = _expand_w(wt, bias, c, o, bg)
    if n_v <= 512:
        return _conv_onehot(h, idx, wx, bias_bo, n_v, c, o, elu, bg)
    g = _sc_gather(h, idx)
    return _conv_mm(g, wx, bias_bo, n_v, c, o, elu, bg)


def _pool(d, hv, tm=256):
    P, Q = d.shape
    BF = hv.shape[1]
    tm = min(tm, _round_up(P, 8))
    grid = (pl.cdiv(P, tm),)

    def body(d_ref, h_ref, o_ref):
        o_ref[...] = lax.dot_general(
            d_ref[...], h_ref[...], (((1,), (0,)), ((), ())),
            preferred_element_type=jnp.float32)

    return pl.pallas_call(
        body,
        grid=grid,
        in_specs=[
            pl.BlockSpec((tm, Q), lambda i: (i, 0)),
            pl.BlockSpec((Q, BF), lambda i: (0, 0)),
        ],
        out_specs=pl.BlockSpec((tm, BF), lambda i: (i, 0)),
        out_shape=jax.ShapeDtypeStruct((P, BF), jnp.float32),
    )(d, hv)


def _mm(a, w, bias):
    M, K = a.shape
    O = w.shape[0]

    def body(a_ref, w_ref, b_ref, o_ref):
        acc = lax.dot_general(
            a_ref[...], w_ref[...], (((1,), (1,)), ((), ())),
            preferred_element_type=jnp.float32)
        o_ref[...] = acc + b_ref[...]

    return pl.pallas_call(
        body,
        in_specs=[pl.BlockSpec((M, K), lambda: (0, 0)),
                  pl.BlockSpec((O, K), lambda: (0, 0)),
                  pl.BlockSpec((1, O), lambda: (0, 0))],
        out_specs=pl.BlockSpec((M, O), lambda: (0, 0)),
        out_shape=jax.ShapeDtypeStruct((M, O), jnp.float32),
    )(a, w, bias.reshape(1, O))


def _mk_idx(spiral, bc, n_v):
    n_pad = _npad(n_v)
    sp = jnp.pad(spiral, ((0, n_pad - n_v), (0, 0)))
    if n_v <= 512:
        return jax.nn.one_hot(sp.T, n_v, dtype=BF16)
    flat = sp.T.reshape(-1)
    m_pad = _round_up(flat.size, NW * _gather_window(bc))
    return jnp.pad(flat, (0, m_pad - flat.size))


def kernel(x, spiral0, spiral1, spiral2, spiral3,
           Wc0, bc0, Wc1, bc1, Wc2, bc2, Wc3, bc3, Wc4, bc4,
           We, be, Wdfc, bdfc,
           Wd0, bd0, Wd1, bd1, Wd2, bd2, Wd3, bd3, Wd4, bd4,
           D0, D1, D2, D3, U0, U1, U2, U3):
    n0, n1, n2, n3, n4 = 5024, 1257, 315, 80, 21
    sp = (spiral0, spiral1, spiral2, spiral3)
    nv = (n0, n1, n2, n3)
    idx = {}

    def sconv(h, lvl, wt, bias, c, o, elu=True):
        key = (lvl, c)
        if key not in idx:
            idx[key] = _mk_idx(sp[lvl], B * c, nv[lvl])
        return _sconv(h, idx[key], wt, bias, nv[lvl], c, o, elu)

    xp = jnp.pad(x, ((0, 0), (0, 0), (0, 1)))
    h = xp.transpose(1, 0, 2).reshape(n0, B * 4)
    w0 = jnp.pad(Wc0.reshape(16, SS, 3), ((0, 0), (0, 0), (0, 1))).reshape(16, SS * 4)
    h = sconv(h, 0, w0, bc0, 4, 16)
    h = sconv(h, 0, Wc1, bc1, 16, 32)
    h = _pool(D0, h)
    h = sconv(h, 1, Wc2, bc2, 32, 64)
    h = _pool(D1, h)
    h = sconv(h, 2, Wc3, bc3, 64, 96)
    h = _pool(D2, h)
    h = sconv(h, 3, Wc4, bc4, 96, 128)
    h = _pool(D3, h)

    hz = h.reshape(n4, B, 128).transpose(1, 0, 2).reshape(B, n4 * 128)
    z = _mm(hz, We, be)
    hd = _mm(z, Wdfc, bdfc)
    h = hd.reshape(B, n4, 128).transpose(1, 0, 2).reshape(n4, B * 128)

    h = _pool(U3, h)
    h = sconv(h, 3, Wd0, bd0, 128, 96)
    h = _pool(U2, h)
    h = sconv(h, 2, Wd1, bd1, 96, 64)
    h = _pool(U1, h)
    h = sconv(h, 1, Wd2, bd2, 64, 32)
    h = _pool(U0, h)
    h = sconv(h, 0, Wd3, bd3, 32, 32)
    h = sconv(h, 0, Wd4, bd4, 32, 3, elu=False)

    return h.reshape(n0, B, 3).transpose(1, 0, 2)


# speedup vs baseline: 1.0807x; 1.0669x over previous
import functools

import jax
import jax.numpy as jnp
from jax import lax
from jax.experimental import pallas as pl
from jax.experimental.pallas import tpu as pltpu
from jax.experimental.pallas import tpu_sc as plsc

B = 32
SS = 12
NW = 32
TN = 256
BF16 = jnp.bfloat16


def _round_up(v, m):
    return (v + m - 1) // m * m


def _npad(n_v):
    return _round_up(n_v, TN)


def _bgroup(c, o):
    for bg in (8, 16):
        if (bg * c) % 128 == 0 and (bg * o) % 128 == 0:
            return bg
    return B


def _gather_window(bc):
    return max(8, min(128, 49152 // bc // 8 * 8))


def _sc_gather(table, idx):
    bc = table.shape[1]
    m = idx.shape[0]
    win = _gather_window(bc)
    chunk = m // NW
    T = chunk // win
    mesh = plsc.VectorSubcoreMesh(core_axis_name="c", subcore_axis_name="s")

    @functools.partial(
        pl.kernel,
        out_type=jax.ShapeDtypeStruct((m, bc), jnp.float32),
        mesh=mesh,
        scratch_types=[
            pltpu.VMEM((chunk,), jnp.int32),
            pltpu.VMEM((win, bc), jnp.float32),
            pltpu.VMEM((win, bc), jnp.float32),
            pltpu.SemaphoreType.DMA,
            pltpu.SemaphoreType.DMA,
            pltpu.SemaphoreType.DMA,
            pltpu.SemaphoreType.DMA,
            pltpu.SemaphoreType.DMA,
        ],
    )
    def k(tab, idx_hbm, out, idx_v, r0, r1, sl, sg0, sg1, so0, so1):
        w = lax.axis_index("s") * 2 + lax.axis_index("c")
        base = w * chunk
        ld = pltpu.make_async_copy(idx_hbm.at[pl.ds(base, chunk)], idx_v, sl)
        ld.start()
        ld.wait()
        rbufs = (r0, r1)
        gsems = (sg0, sg1)
        osems = (so0, so1)

        def gat(t, b):
            return pltpu.make_async_copy(
                tab.at[idx_v.at[pl.ds(t * win, win)]], rbufs[b], gsems[b])

        def halfstep(t, b):
            @pl.when(jnp.logical_and(t >= 1, t + 1 < T)
                     | (t == T) | jnp.logical_and(t == T + 1, T >= 2))
            def _():
                pltpu.make_async_copy(
                    rbufs[1 - b], out.at[pl.ds(0, win)], osems[1 - b]).wait()

            @pl.when(t + 1 < T)
            def _():
                gat(t + 1, 1 - b).start()

            @pl.when(t < T)
            def _():
                gat(t, b).wait()
                pltpu.make_async_copy(
                    rbufs[b], out.at[pl.ds(base + t * win, win)], osems[b]
                ).start()

        @pl.when(T > 0)
        def _():
            gat(0, 0).start()

        def body(i, carry):
            halfstep(2 * i, 0)
            halfstep(2 * i + 1, 1)
            return carry

        lax.fori_loop(0, (T + 3) // 2, body, 0)

    return k(table, idx)


def _conv_mm(g, wx, bias_bo, n_v, c, o, elu, bg):
    n_pad = _npad(n_v)
    gsz = B // bg
    grid = (n_pad // TN, gsz)
    g3 = g.reshape(SS, n_pad, B * c)

    def body(a_ref, w_ref, b_ref, o_ref):
        acc = jnp.zeros((TN, bg * o), jnp.float32)
        for s in range(SS):
            acc += lax.dot_general(
                a_ref[s].astype(BF16), w_ref[s], (((1,), (0,)), ((), ())),
                preferred_element_type=jnp.float32)
        r = acc + b_ref[...]
        if elu:
            r = jnp.where(r > 0, r, jnp.exp(r) - 1.0)
        i = pl.program_id(0)
        vert = i * TN + lax.broadcasted_iota(jnp.int32, (TN, bg * o), 0)
        o_ref[...] = jnp.where(vert < n_v - 1, r, 0.0)

    return pl.pallas_call(
        body,
        grid=grid,
        in_specs=[
            pl.BlockSpec((SS, TN, bg * c), lambda i, gi: (0, i, gi)),
            pl.BlockSpec((SS, bg * c, bg * o), lambda i, gi: (0, 0, 0)),
            pl.BlockSpec((1, bg * o), lambda i, gi: (0, gi)),
        ],
        out_specs=pl.BlockSpec((TN, bg * o), lambda i, gi: (i, gi)),
        out_shape=jax.ShapeDtypeStruct((n_v, B * o), jnp.float32),
    )(g3, wx, bias_bo)


def _expand_w(wt, bias, c, o, bg):
    wsr = wt.reshape(o, SS, c).transpose(1, 2, 0)
    eye = jnp.eye(bg, dtype=wt.dtype)
    wx = jnp.einsum('bB,sco->sbcBo', eye, wsr)
    wx = wx.reshape(SS, bg * c, bg * o).astype(BF16)
    bias_bo = jnp.tile(bias, B).reshape(1, B * o)
    return wx, bias_bo


def _conv_onehot(h, oh, wx, bias_bo, n_v, c, o, elu, bg):
    n_pad = oh.shape[1]
    gsz = B // bg
    grid = (n_pad // TN, gsz)

    def body(oh_ref, h_ref, w_ref, b_ref, o_ref):
        h16 = h_ref[...].astype(BF16)
        acc = jnp.zeros((TN, bg * o), jnp.float32)
        for s in range(SS):
            g_s = lax.dot_general(
                oh_ref[s], h16, (((1,), (0,)), ((), ())),
                preferred_element_type=jnp.float32)
            acc += lax.dot_general(
                g_s.astype(BF16), w_ref[s], (((1,), (0,)), ((), ())),
                preferred_element_type=jnp.float32)
        r = acc + b_ref[...]
        if elu:
            r = jnp.where(r > 0, r, jnp.exp(r) - 1.0)
        i = pl.program_id(0)
        vert = i * TN + lax.broadcasted_iota(jnp.int32, (TN, bg * o), 0)
        o_ref[...] = jnp.where(vert < n_v - 1, r, 0.0)

    return pl.pallas_call(
        body,
        grid=grid,
        in_specs=[
            pl.BlockSpec((SS, TN, n_v), lambda i, gi: (0, i, 0)),
            pl.BlockSpec((n_v, bg * c), lambda i, gi: (0, gi)),
            pl.BlockSpec((SS, bg * c, bg * o), lambda i, gi: (0, 0, 0)),
            pl.BlockSpec((1, bg * o), lambda i, gi: (0, gi)),
        ],
        out_specs=pl.BlockSpec((TN, bg * o), lambda i, gi: (i, gi)),
        out_shape=jax.ShapeDtypeStruct((n_v, B * o), jnp.float32),
    )(oh, h, wx, bias_bo)


def _sconv(h, idx, wt, bias, n_v, c, o, elu=True):
    bg = _bgroup(c, o)
    wx, bias_bo = _expand_w(wt, bias, c, o, bg)
    if n_v <= 512:
        return _conv_onehot(h, idx, wx, bias_bo, n_v, c, o, elu, bg)
    g = _sc_gather(h, idx)
    return _conv_mm(g, wx, bias_bo, n_v, c, o, elu, bg)


def _pool(d, hv, tm=256):
    P, Q = d.shape
    BF = hv.shape[1]
    tm = min(tm, _round_up(P, 8))
    grid = (pl.cdiv(P, tm),)

    def body(d_ref, h_ref, o_ref):
        o_ref[...] = lax.dot_general(
            d_ref[...], h_ref[...], (((1,), (0,)), ((), ())),
            preferred_element_type=jnp.float32)

    return pl.pallas_call(
        body,
        grid=grid,
        in_specs=[
            pl.BlockSpec((tm, Q), lambda i: (i, 0)),
            pl.BlockSpec((Q, BF), lambda i: (0, 0)),
        ],
        out_specs=pl.BlockSpec((tm, BF), lambda i: (i, 0)),
        out_shape=jax.ShapeDtypeStruct((P, BF), jnp.float32),
    )(d, hv)


def _mm(a, w, bias):
    M, K = a.shape
    O = w.shape[0]

    def body(a_ref, w_ref, b_ref, o_ref):
        acc = lax.dot_general(
            a_ref[...], w_ref[...], (((1,), (1,)), ((), ())),
            preferred_element_type=jnp.float32)
        o_ref[...] = acc + b_ref[...]

    return pl.pallas_call(
        body,
        in_specs=[pl.BlockSpec((M, K), lambda: (0, 0)),
                  pl.BlockSpec((O, K), lambda: (0, 0)),
                  pl.BlockSpec((1, O), lambda: (0, 0))],
        out_specs=pl.BlockSpec((M, O), lambda: (0, 0)),
        out_shape=jax.ShapeDtypeStruct((M, O), jnp.float32),
    )(a, w, bias.reshape(1, O))


def _mk_idx(spiral, bc, n_v):
    n_pad = _npad(n_v)
    sp = jnp.pad(spiral, ((0, n_pad - n_v), (0, 0)))
    if n_v <= 512:
        return jax.nn.one_hot(sp.T, n_v, dtype=BF16)
    flat = sp.T.reshape(-1)
    m_pad = _round_up(flat.size, NW * _gather_window(bc))
    return jnp.pad(flat, (0, m_pad - flat.size))


def kernel(x, spiral0, spiral1, spiral2, spiral3,
           Wc0, bc0, Wc1, bc1, Wc2, bc2, Wc3, bc3, Wc4, bc4,
           We, be, Wdfc, bdfc,
           Wd0, bd0, Wd1, bd1, Wd2, bd2, Wd3, bd3, Wd4, bd4,
           D0, D1, D2, D3, U0, U1, U2, U3):
    n0, n1, n2, n3, n4 = 5024, 1257, 315, 80, 21
    sp = (spiral0, spiral1, spiral2, spiral3)
    nv = (n0, n1, n2, n3)
    idx = {}

    def sconv(h, lvl, wt, bias, c, o, elu=True):
        key = (lvl, c)
        if key not in idx:
            idx[key] = _mk_idx(sp[lvl], B * c, nv[lvl])
        return _sconv(h, idx[key], wt, bias, nv[lvl], c, o, elu)

    xp = jnp.pad(x, ((0, 0), (0, 0), (0, 1)))
    h = xp.transpose(1, 0, 2).reshape(n0, B * 4)
    w0 = jnp.pad(Wc0.reshape(16, SS, 3), ((0, 0), (0, 0), (0, 1))).reshape(16, SS * 4)
    h = sconv(h, 0, w0, bc0, 4, 16)
    h = sconv(h, 0, Wc1, bc1, 16, 32)
    h = _pool(D0, h)
    h = sconv(h, 1, Wc2, bc2, 32, 64)
    h = _pool(D1, h)
    h = sconv(h, 2, Wc3, bc3, 64, 96)
    h = _pool(D2, h)
    h = sconv(h, 3, Wc4, bc4, 96, 128)
    h = _pool(D3, h)

    hz = h.reshape(n4, B, 128).transpose(1, 0, 2).reshape(B, n4 * 128)
    z = _mm(hz, We, be)
    hd = _mm(z, Wdfc, bdfc)
    h = hd.reshape(B, n4, 128).transpose(1, 0, 2).reshape(n4, B * 128)

    h = _pool(U3, h)
    h = sconv(h, 3, Wd0, bd0, 128, 96)
    h = _pool(U2, h)
    h = sconv(h, 2, Wd1, bd1, 96, 64)
    h = _pool(U1, h)
    h = sconv(h, 1, Wd2, bd2, 64, 32)
    h = _pool(U0, h)
    h = sconv(h, 0, Wd3, bd3, 32, 32)
    h = sconv(h, 0, Wd4, bd4, 32, 3, elu=False)

    return h.reshape(n0, B, 3).transpose(1, 0, 2)
